# Initial kernel scaffold; baseline (speedup 1.0000x reference)
import jax
import jax.numpy as jnp
from jax.experimental import pallas as pl


def kernel(distance, FEATURE):
    raise NotImplementedError("write your pallas kernel here")


# speedup vs baseline: 3.3579x; 3.3579x over previous
import functools

import jax
import jax.numpy as jnp
from jax import lax
from jax.experimental import pallas as pl
from jax.experimental.pallas import tpu as pltpu
from jax.experimental.pallas import tpu_sc as plsc

NC = 2
NS = 16
NW = NC * NS

N_ROWS, N_COLS = 16384, 26
B = N_ROWS * N_COLS
D = 128
BPW = B // NW
CHUNK = 128
NCHUNK = BPW // CHUNK
NBUF = 6
NGROUP = NCHUNK // NBUF
NTAIL = NCHUNK - NGROUP * NBUF


def _gather_body(idx_hbm, table_hbm, out_hbm, idx_v, rows_v, gsem, ssem):
    cid = lax.axis_index("c")
    sid = lax.axis_index("s")
    wid = sid * NC + cid
    pltpu.sync_copy(idx_hbm.at[wid], idx_v)
    row0 = wid * BPW

    def do_chunks(base_chunk, nbuf):
        copies = []
        for b in range(nbuf):
            copies.append(pltpu.async_copy(
                table_hbm.at[idx_v.at[base_chunk + b]], rows_v.at[b],
                gsem.at[b]))
        stores = []
        for b in range(nbuf):
            copies[b].wait()
            stores.append(pltpu.async_copy(
                rows_v.at[b],
                out_hbm.at[pl.ds(row0 + (base_chunk + b) * CHUNK, CHUNK)],
                ssem.at[b]))
        for b in range(nbuf):
            stores[b].wait()

    def group(g, carry):
        do_chunks(g * NBUF, NBUF)
        return carry

    lax.fori_loop(0, NGROUP, group, 0)
    if NTAIL:
        do_chunks(NGROUP * NBUF, NTAIL)


@functools.partial(jax.jit, static_argnames=())
def _sc_gather(idx, table):
    kern = pl.kernel(
        _gather_body,
        out_type=jax.ShapeDtypeStruct((B, D), jnp.float32),
        mesh=plsc.VectorSubcoreMesh(
            core_axis_name="c", subcore_axis_name="s",
            num_cores=NC, num_subcores=NS),
        scratch_types=[
            pltpu.VMEM((NCHUNK, CHUNK), jnp.int32),
            pltpu.VMEM((NBUF, CHUNK, D), jnp.float32),
            pltpu.SemaphoreType.DMA((NBUF,)),
            pltpu.SemaphoreType.DMA((NBUF,)),
        ],
    )
    return kern(idx, table)


def kernel(distance, FEATURE):
    idx = distance.reshape(NW, NCHUNK, CHUNK).astype(jnp.int32)
    out = _sc_gather(idx, FEATURE)
    return out.reshape(N_ROWS, N_COLS, D)


# speedup vs baseline: 3.4061x; 1.0143x over previous
import functools

import jax
import jax.numpy as jnp
from jax import lax
from jax.experimental import pallas as pl
from jax.experimental.pallas import tpu as pltpu
from jax.experimental.pallas import tpu_sc as plsc

NC = 2
NS = 16
NW = NC * NS

N_ROWS, N_COLS = 16384, 26
B = N_ROWS * N_COLS
D = 128
BPW = B // NW
CHUNK = 104
NCHUNK = BPW // CHUNK
NBUF = 8
NGROUP = NCHUNK // NBUF


def _gather_body(idx_hbm, table_hbm, out_hbm, idx_v, rows_v, gsem, ssem):
    cid = lax.axis_index("c")
    sid = lax.axis_index("s")
    wid = sid * NC + cid
    pltpu.sync_copy(idx_hbm.at[wid], idx_v)
    row0 = wid * BPW

    def fire_gather(chunk, slot):
        return pltpu.async_copy(
            table_hbm.at[idx_v.at[chunk]], rows_v.at[slot], gsem.at[slot])

    def fire_store(chunk, slot):
        return pltpu.async_copy(
            rows_v.at[slot], out_hbm.at[pl.ds(row0 + chunk * CHUNK, CHUNK)],
            ssem.at[slot])

    for b in range(NBUF):
        fire_gather(b, b)

    def group(g, carry):
        for b in range(NBUF):
            j = g * NBUF + b
            pltpu.make_async_copy(
                table_hbm.at[idx_v.at[j]], rows_v.at[b], gsem.at[b]).wait()
            fire_store(j, b).wait()
            fire_gather(j + NBUF, b)
        return carry

    lax.fori_loop(0, NGROUP - 1, group, 0)

    for b in range(NBUF):
        j = (NGROUP - 1) * NBUF + b
        pltpu.make_async_copy(
            table_hbm.at[idx_v.at[j]], rows_v.at[b], gsem.at[b]).wait()
        fire_store(j, b).wait()


@functools.partial(jax.jit, static_argnames=())
def _sc_gather(idx, table):
    kern = pl.kernel(
        _gather_body,
        out_type=jax.ShapeDtypeStruct((B, D), jnp.float32),
        mesh=plsc.VectorSubcoreMesh(
            core_axis_name="c", subcore_axis_name="s",
            num_cores=NC, num_subcores=NS),
        scratch_types=[
            pltpu.VMEM((NCHUNK, CHUNK), jnp.int32),
            pltpu.VMEM((NBUF, CHUNK, D), jnp.float32),
            pltpu.SemaphoreType.DMA((NBUF,)),
            pltpu.SemaphoreType.DMA((NBUF,)),
        ],
    )
    return kern(idx, table)


def kernel(distance, FEATURE):
    idx = distance.reshape(NW, NCHUNK, CHUNK).astype(jnp.int32)
    out = _sc_gather(idx, FEATURE)
    return out.reshape(N_ROWS, N_COLS, D)


# speedup vs baseline: 11.7878x; 3.4608x over previous
import functools

import jax
import jax.numpy as jnp
from jax import lax
from jax.experimental import pallas as pl
from jax.experimental.pallas import tpu as pltpu
from jax.experimental.pallas import tpu_sc as plsc

NC = 2
NS = 16
NW = NC * NS

N_ROWS, N_COLS = 16384, 26
B = N_ROWS * N_COLS
D = 128
BPW = B // NW
CHUNK = 104
NCHUNK = BPW // CHUNK
NBUF = 8
NGROUP = NCHUNK // NBUF


def _gather_body(idx_hbm, table_hbm, out_hbm, idx_v, rows_v, gsem, ssem):
    cid = lax.axis_index("c")
    sid = lax.axis_index("s")
    wid = sid * NC + cid
    pltpu.sync_copy(idx_hbm.at[wid], idx_v)
    row0 = wid * BPW

    def fire_gather(chunk, slot):
        return pltpu.async_copy(
            table_hbm.at[idx_v.at[chunk]], rows_v.at[slot], gsem.at[slot])

    def fire_store(chunk, slot):
        return pltpu.async_copy(
            rows_v.at[slot], out_hbm.at[pl.ds(row0 + chunk * CHUNK, CHUNK)],
            ssem.at[slot])

    for b in range(NBUF):
        fire_gather(b, b)

    def group(g, carry):
        for b in range(NBUF):
            j = g * NBUF + b
            pltpu.make_async_copy(
                table_hbm.at[idx_v.at[j]], rows_v.at[b], gsem.at[b]).wait()
            fire_store(j, b).wait()
            fire_gather(j + NBUF, b)
        return carry

    lax.fori_loop(0, NGROUP - 1, group, 0)

    for b in range(NBUF):
        j = (NGROUP - 1) * NBUF + b
        pltpu.make_async_copy(
            table_hbm.at[idx_v.at[j]], rows_v.at[b], gsem.at[b]).wait()
        fire_store(j, b).wait()


@functools.partial(jax.jit, static_argnames=())
def _sc_gather(idx, table):
    kern = pl.kernel(
        _gather_body,
        out_type=jax.ShapeDtypeStruct((B, D), jnp.float32),
        mesh=plsc.VectorSubcoreMesh(
            core_axis_name="c", subcore_axis_name="s",
            num_cores=NC, num_subcores=NS),
        scratch_types=[
            pltpu.VMEM((NCHUNK, CHUNK), jnp.int32),
            pltpu.VMEM((NBUF, CHUNK, D), jnp.float32),
            pltpu.SemaphoreType.DMA((NBUF,)),
            pltpu.SemaphoreType.DMA((NBUF,)),
        ],
    )
    return kern(idx, table)


def kernel(distance, FEATURE):
    idx = jnp.transpose(distance).reshape(NW, NCHUNK, CHUNK).astype(jnp.int32)
    out = _sc_gather(idx, FEATURE)
    return out.reshape(N_COLS, N_ROWS, D).transpose(1, 0, 2)


# speedup vs baseline: 11.8367x; 1.0041x over previous
import functools

import jax
import jax.numpy as jnp
from jax import lax
from jax.experimental import pallas as pl
from jax.experimental.pallas import tpu as pltpu
from jax.experimental.pallas import tpu_sc as plsc

NC = 2
NS = 16
NW = NC * NS

N_ROWS, N_COLS = 16384, 26
B = N_ROWS * N_COLS
D = 128
BPW = B // NW
CHUNK = 104
NCHUNK = BPW // CHUNK
NBUF = 8
NGROUP = NCHUNK // NBUF


def _gather_body(idx_hbm, table_hbm, out_hbm, idx_v, rows_v, gsem, ssem):
    cid = lax.axis_index("c")
    sid = lax.axis_index("s")
    wid = sid * NC + cid
    pltpu.sync_copy(idx_hbm.at[wid], idx_v)
    row0 = wid * BPW

    def fire_gather(chunk, slot):
        return pltpu.async_copy(
            table_hbm.at[idx_v.at[chunk]], rows_v.at[slot], gsem.at[slot])

    def fire_store(chunk, slot):
        return pltpu.async_copy(
            rows_v.at[slot], out_hbm.at[pl.ds(row0 + chunk * CHUNK, CHUNK)],
            ssem.at[slot])

    def wait_gather(chunk, slot):
        pltpu.make_async_copy(
            table_hbm.at[idx_v.at[chunk]], rows_v.at[slot], gsem.at[slot]).wait()

    def wait_store(chunk, slot):
        pltpu.make_async_copy(
            rows_v.at[slot],
            out_hbm.at[pl.ds(row0 + chunk * CHUNK, CHUNK)], ssem.at[slot]).wait()

    LA = NBUF - 2
    for c in range(LA):
        fire_gather(c, c)
    for j in range(2):
        wait_gather(j, j)
        fire_store(j, j)
        fire_gather(j + LA, j + LA)

    def group(g, carry):
        for k in range(NBUF):
            j = 2 + g * NBUF + k
            b = (2 + k) % NBUF
            s2 = (b + LA) % NBUF
            wait_gather(j, b)
            fire_store(j, b)
            wait_store(j - 2, s2)
            fire_gather(j + LA, s2)
        return carry

    lax.fori_loop(0, (NCHUNK - 2 - LA) // NBUF, group, 0)

    for j in range(NCHUNK - LA, NCHUNK):
        b = j % NBUF
        wait_gather(j, b)
        fire_store(j, b)
        wait_store(j - 2, (b + LA) % NBUF)
    for j in range(NCHUNK - 2, NCHUNK):
        wait_store(j, j % NBUF)


@functools.partial(jax.jit, static_argnames=())
def _sc_gather(idx, table):
    kern = pl.kernel(
        _gather_body,
        out_type=jax.ShapeDtypeStruct((B, D), jnp.float32),
        mesh=plsc.VectorSubcoreMesh(
            core_axis_name="c", subcore_axis_name="s",
            num_cores=NC, num_subcores=NS),
        scratch_types=[
            pltpu.VMEM((NCHUNK, CHUNK), jnp.int32),
            pltpu.VMEM((NBUF, CHUNK, D), jnp.float32),
            pltpu.SemaphoreType.DMA((NBUF,)),
            pltpu.SemaphoreType.DMA((NBUF,)),
        ],
    )
    return kern(idx, table)


def kernel(distance, FEATURE):
    idx = jnp.transpose(distance).reshape(NW, NCHUNK, CHUNK).astype(jnp.int32)
    out = _sc_gather(idx, FEATURE)
    return out.reshape(N_COLS, N_ROWS, D).transpose(1, 0, 2)


# speedup vs baseline: 11.9784x; 1.0120x over previous
import functools

import jax
import jax.numpy as jnp
from jax import lax
from jax.experimental import pallas as pl
from jax.experimental.pallas import tpu as pltpu
from jax.experimental.pallas import tpu_sc as plsc

NC = 2
NS = 16
NW = NC * NS

N_ROWS, N_COLS = 16384, 26
B = N_ROWS * N_COLS
D = 128
BPW = B // NW
CHUNK = 128
NCHUNK = BPW // CHUNK
NBLK = B // CHUNK
NBUF = 6
LA = NBUF - 2


def _gather_body(idx_hbm, table_hbm, out_hbm, idx_v, rows_v, gsem, ssem):
    cid = lax.axis_index("c")
    sid = lax.axis_index("s")
    wid = sid * NC + cid
    pltpu.sync_copy(idx_hbm.at[wid], idx_v)
    blk0 = wid * NCHUNK

    def fire_gather(chunk, slot):
        return pltpu.async_copy(
            table_hbm.at[idx_v.at[chunk]], rows_v.at[slot, 0], gsem.at[slot])

    def fire_store(chunk, slot):
        return pltpu.async_copy(
            rows_v.at[slot], out_hbm.at[pl.ds(blk0 + chunk, 1)],
            ssem.at[slot])

    def wait_gather(chunk, slot):
        pltpu.make_async_copy(
            table_hbm.at[idx_v.at[chunk]], rows_v.at[slot, 0], gsem.at[slot]).wait()

    def wait_store(chunk, slot):
        pltpu.make_async_copy(
            rows_v.at[slot], out_hbm.at[pl.ds(blk0 + chunk, 1)],
            ssem.at[slot]).wait()

    for c in range(LA):
        fire_gather(c, c)
    for j in range(2):
        wait_gather(j, j)
        fire_store(j, j)
        fire_gather(j + LA, (j + LA) % NBUF)

    def group(g, carry):
        for k in range(NBUF):
            j = 2 + g * NBUF + k
            b = (2 + k) % NBUF
            s2 = (b + LA) % NBUF
            wait_gather(j, b)
            fire_store(j, b)
            wait_store(j - 2, s2)
            fire_gather(j + LA, s2)
        return carry

    G = (NCHUNK - 2 - LA) // NBUF
    lax.fori_loop(0, G, group, 0)

    for j in range(2 + G * NBUF, NCHUNK - LA):
        b = j % NBUF
        s2 = (b + LA) % NBUF
        wait_gather(j, b)
        fire_store(j, b)
        wait_store(j - 2, s2)
        fire_gather(j + LA, s2)
    for j in range(NCHUNK - LA, NCHUNK):
        b = j % NBUF
        wait_gather(j, b)
        fire_store(j, b)
        wait_store(j - 2, (b + LA) % NBUF)
    for j in range(NCHUNK - 2, NCHUNK):
        wait_store(j, j % NBUF)


@functools.partial(jax.jit, static_argnames=())
def _sc_gather(idx, table):
    kern = pl.kernel(
        _gather_body,
        out_type=jax.ShapeDtypeStruct((NBLK, CHUNK, D), jnp.float32),
        mesh=plsc.VectorSubcoreMesh(
            core_axis_name="c", subcore_axis_name="s",
            num_cores=NC, num_subcores=NS),
        scratch_types=[
            pltpu.VMEM((NCHUNK, CHUNK), jnp.int32),
            pltpu.VMEM((NBUF, 1, CHUNK, D), jnp.float32),
            pltpu.SemaphoreType.DMA((NBUF,)),
            pltpu.SemaphoreType.DMA((NBUF,)),
        ],
    )
    return kern(idx, table)


def kernel(distance, FEATURE):
    idx = jnp.transpose(distance).reshape(NW, NCHUNK, CHUNK).astype(jnp.int32)
    out = _sc_gather(idx, FEATURE)
    return out.reshape(N_COLS, N_ROWS, D).transpose(1, 0, 2)
